# baseline (speedup 1.0000x reference)
import functools

import jax
import jax.numpy as jnp
from jax import lax
from jax.experimental import pallas as pl
from jax.experimental.pallas import tpu as pltpu

_PLANES_PER_STEP = 33
_NUM_CHUNKS = 3
_CHUNK = _PLANES_PER_STEP // _NUM_CHUNKS


def _make_body(n, d, grid):
    def _chunk_copy(scr, out_hbm, sems, g, slot, q):
        rows = g * _PLANES_PER_STEP + q * _CHUNK
        return pltpu.make_async_copy(
            scr.at[slot, pl.ds(q * _CHUNK, _CHUNK)],
            out_hbm.at[pl.ds(rows, _CHUNK)],
            sems.at[slot, q],
        )

    def _body(yyt_ref, out_hbm, scr, sems):
        g = pl.program_id(0)
        slot = lax.rem(g, 2)

        @pl.when(g >= 2)
        def _reclaim():
            for q in range(_NUM_CHUNKS):
                _chunk_copy(scr, out_hbm, sems, g - 2, slot, q).wait()

        j0 = g * _PLANES_PER_STEP
        rolled = pltpu.roll(yyt_ref[:], 2 * n - 1 - j0, axis=1)
        for jj in range(_PLANES_PER_STEP):
            scr[slot, jj] = rolled[:, jj : jj + n]

        for q in range(_NUM_CHUNKS):
            _chunk_copy(scr, out_hbm, sems, g, slot, q).start()

        @pl.when(g == grid - 1)
        def _drain():
            for q in range(_NUM_CHUNKS):
                _chunk_copy(scr, out_hbm, sems, g - 1, 1 - slot, q).wait()
                _chunk_copy(scr, out_hbm, sems, g, slot, q).wait()

    return _body


def kernel(a, b, c, y):
    n, d = y.shape
    yt = y.T
    yyt = jnp.concatenate([yt, yt], axis=1)
    num_planes = n - 1
    grid = pl.cdiv(num_planes, _PLANES_PER_STEP)
    run = pl.pallas_call(
        _make_body(n, d, grid),
        grid=(grid,),
        in_specs=[pl.BlockSpec((d, 2 * n), lambda g: (0, 0))],
        out_specs=pl.BlockSpec(memory_space=pl.ANY),
        out_shape=jax.ShapeDtypeStruct((num_planes, d, n), jnp.float32),
        scratch_shapes=[
            pltpu.VMEM((2, _PLANES_PER_STEP, d, n), jnp.float32),
            pltpu.SemaphoreType.DMA((2, _NUM_CHUNKS)),
        ],
    )
    t = run(yyt)
    return jnp.transpose(t, (2, 0, 1))


# speedup vs baseline: 1.1101x; 1.1101x over previous
import functools

import jax
import jax.numpy as jnp
from jax.experimental import pallas as pl
from jax.experimental.pallas import tpu as pltpu

_PLANES_PER_STEP = 128


def _make_body(n, d):
    def _body(yyt_ref, out_ref):
        j0 = pl.program_id(0) * _PLANES_PER_STEP
        rolled = pltpu.roll(yyt_ref[:], 2 * n - 1 - j0, axis=1)
        for jj in range(_PLANES_PER_STEP):
            out_ref[jj] = rolled[:, jj : jj + n]

    return _body


def kernel(a, b, c, y):
    n, d = y.shape
    yt = y.T
    yyt = jnp.concatenate([yt, yt], axis=1)
    num_planes = n - 1
    grid = pl.cdiv(num_planes, _PLANES_PER_STEP)
    run = pl.pallas_call(
        _make_body(n, d),
        grid=(grid,),
        in_specs=[pl.BlockSpec((d, 2 * n), lambda g: (0, 0))],
        out_specs=pl.BlockSpec((_PLANES_PER_STEP, d, n), lambda g: (g, 0, 0)),
        out_shape=jax.ShapeDtypeStruct((num_planes, d, n), jnp.float32),
    )
    t = run(yyt)
    return jnp.transpose(t, (2, 0, 1))


# speedup vs baseline: 1.1116x; 1.0014x over previous
import jax
import jax.numpy as jnp
from jax.experimental import pallas as pl
from jax.experimental.pallas import tpu as pltpu

_PLANES_PER_STEP = 128


def _make_body(n, d):
    def _body(yyt_ref, out_ref):
        j0 = pl.program_id(0) * _PLANES_PER_STEP
        rolled = pltpu.roll(yyt_ref[:], 2 * n - 1 - j0, axis=1)
        for jj in range(_PLANES_PER_STEP):
            out_ref[jj] = rolled[:, jj : jj + n]

    return _body


def kernel(a, b, c, y):
    n, d = y.shape
    yt = y.T
    yyt = jnp.concatenate([yt, yt], axis=1)
    num_planes = n - 1
    grid = pl.cdiv(num_planes, _PLANES_PER_STEP)
    run = pl.pallas_call(
        _make_body(n, d),
        grid=(grid,),
        in_specs=[pl.BlockSpec((d, 2 * n), lambda g: (0, 0))],
        out_specs=pl.BlockSpec((_PLANES_PER_STEP, d, n), lambda g: (g, 0, 0)),
        out_shape=jax.ShapeDtypeStruct((num_planes, d, n), jnp.float32),
    )
    t = run(yyt)
    return jnp.transpose(t, (2, 0, 1))
